# Initial kernel scaffold; baseline (speedup 1.0000x reference)
import jax
import jax.numpy as jnp
from jax.experimental import pallas as pl


def kernel(data, indices, axis):
    raise NotImplementedError("write your pallas kernel here")


# speedup vs baseline: 1.0880x; 1.0880x over previous
import functools

import jax
import jax.numpy as jnp
from jax import lax
from jax.experimental import pallas as pl
from jax.experimental.pallas import tpu as pltpu
from jax.experimental.pallas import tpu_sc as plsc

_INFO = plsc.get_sparse_core_info()
_NC = _INFO.num_cores
_NS = _INFO.num_subcores
_NW = _NC * _NS
_L = _INFO.num_lanes


def _gather_kernel_body(V, B, D, C, table_hbm, idx_hbm, out_hbm,
                        idx_v, rows_v, sem):
    b_per_w = B // _NW
    wid = lax.axis_index("s") * _NC + lax.axis_index("c")
    base = wid * b_per_w
    n_chunks = b_per_w // C

    def chunk_body(g, _):
        off = base + g * C
        pltpu.sync_copy(idx_hbm.at[pl.ds(off, C)], idx_v)

        def norm_body(i, _):
            v = idx_v[pl.ds(i * _L, _L)]
            idx_v[pl.ds(i * _L, _L)] = jnp.where(v < 0, v + V, v)
            return 0

        lax.fori_loop(0, C // _L, norm_body, 0)
        pltpu.async_copy(table_hbm.at[idx_v], rows_v, sem).wait()
        pltpu.sync_copy(rows_v, out_hbm.at[pl.ds(off, C)])
        return 0

    lax.fori_loop(0, n_chunks, chunk_body, 0)


def kernel(data, indices, axis=0):
    V, D = data.shape
    out_shape = tuple(indices.shape) + (D,)
    B = 1
    for s in indices.shape:
        B *= s
    idx_flat = indices.reshape(B).astype(jnp.int32)

    C = 1024
    assert B % (_NW * C) == 0

    mesh = plsc.VectorSubcoreMesh(core_axis_name="c", subcore_axis_name="s")
    k = functools.partial(
        pl.kernel,
        mesh=mesh,
        compiler_params=pltpu.CompilerParams(use_tc_tiling_on_sc=False),
        out_type=jax.ShapeDtypeStruct((B, D), jnp.float32),
        scratch_types=[
            pltpu.VMEM((C,), jnp.int32),
            pltpu.VMEM((C, D), jnp.float32),
            pltpu.SemaphoreType.DMA,
        ],
    )(functools.partial(_gather_kernel_body, V, B, D, C))
    out = k(data, idx_flat)
    return out.reshape(out_shape)


# speedup vs baseline: 1.1035x; 1.0142x over previous
import functools

import jax
import jax.numpy as jnp
from jax import lax
from jax.experimental import pallas as pl
from jax.experimental.pallas import tpu as pltpu
from jax.experimental.pallas import tpu_sc as plsc

_INFO = plsc.get_sparse_core_info()
_NC = _INFO.num_cores
_NS = _INFO.num_subcores
_NW = _NC * _NS
_L = _INFO.num_lanes

_C = 800
_NBUF = 4


def _gather_kernel_body(V, B, D, table_hbm, idx_hbm, out_hbm,
                        idx_all, rows0, rows1, rows2, rows3,
                        g0, g1, g2, g3, w0, w1, w2, w3):
    rows = (rows0, rows1, rows2, rows3)
    gsem = (g0, g1, g2, g3)
    wsem = (w0, w1, w2, w3)
    b_per_w = B // _NW
    n_chunks = b_per_w // _C
    wid = lax.axis_index("s") * _NC + lax.axis_index("c")
    base = wid * b_per_w

    pltpu.sync_copy(idx_hbm.at[pl.ds(base, b_per_w)], idx_all)

    def norm_body(i, _):
        v = idx_all[pl.ds(i * _L, _L)]
        idx_all[pl.ds(i * _L, _L)] = jnp.where(v < 0, v + V, v)
        return 0

    lax.fori_loop(0, b_per_w // _L, norm_body, 0)

    def fire_gather(g, b):
        pltpu.async_copy(
            table_hbm.at[idx_all.at[pl.ds(g * _C, _C)]], rows[b], gsem[b])

    def wait_bytes(sem, b):
        pltpu.make_async_copy(table_hbm.at[pl.ds(0, _C)], rows[b], sem).wait()

    for b in range(_NBUF):
        fire_gather(b, b)

    def group_body(i, _):
        o = i * _NBUF
        for b in range(_NBUF):
            wait_bytes(gsem[b], b)
            pltpu.async_copy(
                rows[b], out_hbm.at[pl.ds(base + (o + b) * _C, _C)], wsem[b])
        for b in range(_NBUF):
            wait_bytes(wsem[b], b)

            @pl.when(o + _NBUF < n_chunks)
            def _():
                fire_gather(o + _NBUF + b, b)

        return 0

    lax.fori_loop(0, n_chunks // _NBUF, group_body, 0)


def kernel(data, indices, axis=0):
    V, D = data.shape
    out_shape = tuple(indices.shape) + (D,)
    B = 1
    for s in indices.shape:
        B *= s
    idx_flat = indices.reshape(B).astype(jnp.int32)

    b_per_w = B // _NW
    assert B % _NW == 0 and b_per_w % (_C * _NBUF) == 0

    mesh = plsc.VectorSubcoreMesh(core_axis_name="c", subcore_axis_name="s")
    k = functools.partial(
        pl.kernel,
        mesh=mesh,
        compiler_params=pltpu.CompilerParams(use_tc_tiling_on_sc=False),
        out_type=jax.ShapeDtypeStruct((B, D), jnp.float32),
        scratch_types=[
            pltpu.VMEM((b_per_w,), jnp.int32),
        ] + [pltpu.VMEM((_C, D), jnp.float32) for _ in range(_NBUF)]
          + [pltpu.SemaphoreType.DMA for _ in range(2 * _NBUF)],
    )(functools.partial(_gather_kernel_body, V, B, D))
    out = k(data, idx_flat)
    return out.reshape(out_shape)


# speedup vs baseline: 1.7913x; 1.6233x over previous
import functools

import jax
import jax.numpy as jnp
from jax import lax
from jax.experimental import pallas as pl
from jax.experimental.pallas import tpu as pltpu
from jax.experimental.pallas import tpu_sc as plsc

_INFO = plsc.get_sparse_core_info()
_NC = _INFO.num_cores
_NS = _INFO.num_subcores
_NW = _NC * _NS
_L = _INFO.num_lanes

_NBUF = 5


def _gather_kernel_body(V, NB, NS_, D, table_hbm, idxT_hbm, out_hbm,
                        idx_v, b0, b1, b2, b3, b4,
                        g0, g1, g2, g3, g4, w0, w1, w2, w3, w4):
    bufs = (b0, b1, b2, b3, b4)
    gsem = (g0, g1, g2, g3, g4)
    wsem = (w0, w1, w2, w3, w4)
    b_per_w = NB // _NW
    wid = lax.axis_index("s") * _NC + lax.axis_index("c")
    base_b = wid * b_per_w

    pltpu.sync_copy(
        idxT_hbm.at[pl.ds(0, NS_), pl.ds(base_b, b_per_w)], idx_v)

    def norm_body(i, _):
        s = i // (b_per_w // _L)
        c = i % (b_per_w // _L)
        v = idx_v[s, pl.ds(c * _L, _L)]
        idx_v[s, pl.ds(c * _L, _L)] = jnp.where(v < 0, v + V, v)
        return 0

    lax.fori_loop(0, NS_ * (b_per_w // _L), norm_body, 0)

    def fire_gather(s, b):
        pltpu.async_copy(table_hbm.at[idx_v.at[s]], bufs[b], gsem[b])

    def wait_bytes(sem, b):
        pltpu.make_async_copy(
            table_hbm.at[pl.ds(0, b_per_w)], bufs[b], sem).wait()

    for b in range(_NBUF):
        fire_gather(b, b)

    n_groups = NS_ // _NBUF

    def group_body(g, _):
        o = g * _NBUF
        for b in range(_NBUF):
            wait_bytes(gsem[b], b)
            pltpu.async_copy(
                bufs[b], out_hbm.at[pl.ds(base_b, b_per_w), o + b], wsem[b])
        for b in range(_NBUF):
            wait_bytes(wsem[b], b)

            @pl.when(g < n_groups - 1)
            def _():
                fire_gather(o + _NBUF + b, b)

        return 0

    lax.fori_loop(0, n_groups, group_body, 0)


def kernel(data, indices, axis=0):
    V, D = data.shape
    NB, NS_ = indices.shape
    idxT = indices.T.astype(jnp.int32)

    b_per_w = NB // _NW
    assert NB % _NW == 0 and NS_ % _NBUF == 0 and b_per_w % _L == 0

    mesh = plsc.VectorSubcoreMesh(core_axis_name="c", subcore_axis_name="s")
    k = functools.partial(
        pl.kernel,
        mesh=mesh,
        compiler_params=pltpu.CompilerParams(use_tc_tiling_on_sc=False),
        out_type=jax.ShapeDtypeStruct((NB, NS_, D), jnp.float32),
        scratch_types=[
            pltpu.VMEM((NS_, b_per_w), jnp.int32),
        ] + [pltpu.VMEM((b_per_w, D), jnp.float32) for _ in range(_NBUF)]
          + [pltpu.SemaphoreType.DMA for _ in range(2 * _NBUF)],
    )(functools.partial(_gather_kernel_body, V, NB, NS_, D))
    return k(data, idxT)
